# Initial kernel scaffold; baseline (speedup 1.0000x reference)
import jax
import jax.numpy as jnp
from jax.experimental import pallas as pl


def kernel(x, edge_index, edge_type, mlp_w1, mlp_b1, mlp_w2, mlp_b2, mlp_w3, mlp_b3, l0_w0, l0_b0, l0_wl, l0_bl, l0_w1, l0_b1, l1_w0, l1_b0, l1_wl, l1_bl, l1_w1, l1_b1, fc1_w, fc1_b, fc2_w, fc2_b):
    raise NotImplementedError("write your pallas kernel here")


# speedup vs baseline: 6.1363x; 6.1363x over previous
import functools

import jax
import jax.numpy as jnp
from jax import lax
from jax.experimental import pallas as pl
from jax.experimental.pallas import tpu as pltpu
from jax.experimental.pallas import tpu_sc as plsc

N = 10000
E = 320000
D_IN = 128
H = 64

NC = 2
NS = 16
LANES = 16

ROWS_BLK = 1000


def _make_seg_sum(n_rows: int, d: int, rel: int, n_edges: int):
    nsh = n_rows // NC
    ept = n_edges // NS
    n_seg = 5
    seg = ept // n_seg
    assert seg * n_seg == ept and seg % LANES == 0
    cbuf_len = seg + 144
    rows_per_tile = 320
    nrp = rows_per_tile * NS
    assert nrp >= nsh
    zrows = 64
    n_zcopies = rows_per_tile // zrows
    garbage_row = nrp

    mesh = plsc.VectorSubcoreMesh(
        core_axis_name="c", subcore_axis_name="s",
        num_cores=NC, num_subcores=NS)

    @functools.partial(
        pl.kernel,
        out_type=jax.ShapeDtypeStruct((NC, nrp, d), jnp.float32),
        mesh=mesh,
        compiler_params=pltpu.CompilerParams(
            needs_layout_passes=False, use_tc_tiling_on_sc=False),
        scratch_types=dict(
            aggr=pltpu.VMEM_SHARED((nrp + 8, d), jnp.float32),
            src_v=pltpu.VMEM((seg,), jnp.int32),
            dst_v=pltpu.VMEM((seg,), jnp.int32),
            typ_v=pltpu.VMEM((seg,), jnp.int32),
            csrc=pltpu.VMEM((cbuf_len,), jnp.int32),
            cdst=pltpu.VMEM((cbuf_len,), jnp.int32),
            didx=pltpu.VMEM((128,), jnp.int32),
            sidx=pltpu.VMEM((1, 128), jnp.int32),
            gbuf=pltpu.VMEM((128, d), jnp.float32),
            zbuf=pltpu.VMEM((zrows, d), jnp.float32),
            sem=pltpu.SemaphoreType.DMA,
        ),
    )
    def seg_sum(h_hbm, esrc_hbm, edst_hbm, et_hbm, out_hbm,
                aggr, src_v, dst_v, typ_v, csrc, cdst, didx, sidx, gbuf,
                zbuf, sem):
        c = lax.axis_index("c")
        s = lax.axis_index("s")
        lanes_per_row = d // LANES
        lo = c * nsh
        lov = lax.broadcast_in_dim(lo, (LANES,), ())
        hiv = lov + nsh

        def zb(k, _):
            zbuf[k // lanes_per_row,
                 pl.ds((k % lanes_per_row) * LANES, LANES)] = (
                     jnp.zeros((LANES,), jnp.float32))
            return 0
        lax.fori_loop(0, zrows * lanes_per_row, zb, 0)
        row_lo = s * rows_per_tile
        for z in range(n_zcopies):
            pltpu.sync_copy(zbuf, aggr.at[pl.ds(row_lo + z * zrows, zrows)])

        plsc.subcore_barrier()

        for g in range(n_seg):
            ebase = s * ept + g * seg
            pltpu.sync_copy(esrc_hbm.at[pl.ds(ebase, seg)], src_v)
            pltpu.sync_copy(edst_hbm.at[pl.ds(ebase, seg)], dst_v)
            pltpu.sync_copy(et_hbm.at[pl.ds(ebase, seg)], typ_v)

            def cbody(i, off):
                b = i * LANES
                sv = src_v[pl.ds(b, LANES)]
                dv = dst_v[pl.ds(b, LANES)]
                tv = typ_v[pl.ds(b, LANES)]
                m = (tv == rel) & (sv >= lov) & (sv < hiv)
                mi = m.astype(jnp.int32)
                offv = lax.broadcast_in_dim(off, (LANES,), ())
                pos = offv + plsc.cumsum(mi) - mi
                plsc.store_scatter(csrc, [pos], sv - lov, mask=m)
                plsc.store_scatter(cdst, [pos], dv, mask=m)
                return off + jnp.sum(mi)
            off = lax.fori_loop(0, seg // LANES, cbody, jnp.int32(0))

            for j in range(8):
                pidx = off + j * LANES + lax.iota(jnp.int32, 16)
                plsc.store_scatter(csrc, [pidx],
                                   jnp.full((LANES,), garbage_row, jnp.int32))
                plsc.store_scatter(cdst, [pidx],
                                   jnp.zeros((LANES,), jnp.int32))

            nch = (off + 127) // 128

            def chbody(ch, _):
                base = ch * 128
                for j in range(8):
                    sidx[0, pl.ds(j * LANES, LANES)] = (
                        csrc[pl.ds(base + j * LANES, LANES)])
                    didx[pl.ds(j * LANES, LANES)] = (
                        cdst[pl.ds(base + j * LANES, LANES)])
                pltpu.async_copy(h_hbm.at[didx], gbuf, sem).wait()
                pltpu.sync_copy(gbuf, aggr.at[sidx.at[0]], add=True)
                return 0
            lax.fori_loop(0, nch, chbody, 0)

        plsc.subcore_barrier()

        pltpu.sync_copy(aggr.at[pl.ds(row_lo, rows_per_tile)],
                        out_hbm.at[c, pl.ds(row_lo, rows_per_tile)])

    return seg_sum


def _dot(a, b):
    return jnp.dot(a, b, preferred_element_type=jnp.float32)


def _full(shape):
    return pl.BlockSpec(shape, lambda i: (0,) * len(shape))


def _rows(shape):
    return pl.BlockSpec(shape, lambda i: (i,) + (0,) * (len(shape) - 1))


def _mlp_kernel(x_ref, w1_ref, b1_ref, w2_ref, b2_ref, w3_ref, b3_ref,
                w01_ref, b01_ref, h_ref, s0_ref):
    h1 = jnp.maximum(_dot(x_ref[...], w1_ref[...]) + b1_ref[...], 0.0)
    h2 = jnp.maximum(_dot(h1, w2_ref[...]) + b2_ref[...], 0.0)
    hh = _dot(h2, w3_ref[...]) + b3_ref[...]
    h_ref[...] = hh
    s0_ref[...] = _dot(hh, w01_ref[...]) + b01_ref[...]


def _mid_kernel(p0_ref, s0_ref, wl_ref, w11_ref, b11_ref, emb1_ref, s1_ref):
    aggr = p0_ref[0]
    emb1 = jnp.maximum(_dot(aggr, wl_ref[...]) + s0_ref[...], 0.0)
    emb1_ref[...] = emb1
    s1_ref[...] = _dot(emb1, w11_ref[...]) + b11_ref[...]


def _post_kernel(p1_ref, s1_ref, wl_ref, fc1w_ref, fc1b_ref, fc2w_ref,
                 fc2b_ref, out_ref):
    aggr = p1_ref[0]
    emb2 = jnp.maximum(_dot(aggr, wl_ref[...]) + s1_ref[...], 0.0)
    o1 = jnp.maximum(_dot(emb2, fc1w_ref[...]) + fc1b_ref[...], 0.0)
    o2 = _dot(o1, fc2w_ref[...]) + fc2b_ref[...]
    mx = jnp.max(o2, axis=1, keepdims=True)
    lse = jnp.log(jnp.sum(jnp.exp(o2 - mx), axis=1, keepdims=True)) + mx
    out_ref[...] = o2 - lse


def kernel(x, edge_index, edge_type,
           mlp_w1, mlp_b1, mlp_w2, mlp_b2, mlp_w3, mlp_b3,
           l0_w0, l0_b0, l0_wl, l0_bl, l0_w1, l0_b1,
           l1_w0, l1_b0, l1_wl, l1_bl, l1_w1, l1_b1,
           fc1_w, fc1_b, fc2_w, fc2_b):
    n = x.shape[0]
    grid = (n // ROWS_BLK,)

    w01 = l0_w0 + l0_w1
    b01 = (l0_b0 + l0_b1 + l0_bl)[None, :]
    w11 = l1_w0 + l1_w1
    b11 = (l1_b0 + l1_b1 + l1_bl)[None, :]
    fc1_eff = fc1_w[:H] + fc1_w[H:]

    h, s0 = pl.pallas_call(
        _mlp_kernel,
        grid=grid,
        in_specs=[
            _rows((ROWS_BLK, D_IN)),
            _full((D_IN, H)), _full((1, H)),
            _full((H, H)), _full((1, H)),
            _full((H, 2 * H)), _full((1, 2 * H)),
            _full((2 * H, H)), _full((1, H)),
        ],
        out_specs=[_rows((ROWS_BLK, 2 * H)), _rows((ROWS_BLK, H))],
        out_shape=[
            jax.ShapeDtypeStruct((n, 2 * H), jnp.float32),
            jax.ShapeDtypeStruct((n, H), jnp.float32),
        ],
    )(x, mlp_w1, mlp_b1[None, :], mlp_w2, mlp_b2[None, :],
      mlp_w3, mlp_b3[None, :], w01, b01)

    esrc = edge_index[0]
    edst = edge_index[1]
    p0 = _make_seg_sum(n, 2 * H, 2, edge_type.shape[0])(
        h, esrc, edst, edge_type)

    emb1, s1 = pl.pallas_call(
        _mid_kernel,
        grid=grid,
        in_specs=[
            pl.BlockSpec((1, ROWS_BLK, 2 * H), lambda i: (i // 5, i % 5, 0)),
            _rows((ROWS_BLK, H)),
            _full((2 * H, H)),
            _full((H, H)), _full((1, H)),
        ],
        out_specs=[_rows((ROWS_BLK, H)), _rows((ROWS_BLK, H))],
        out_shape=[
            jax.ShapeDtypeStruct((n, H), jnp.float32),
            jax.ShapeDtypeStruct((n, H), jnp.float32),
        ],
    )(p0, s0, l0_wl, w11, b11)

    p1 = _make_seg_sum(n, H, 3, edge_type.shape[0])(
        emb1, esrc, edst, edge_type)

    out = pl.pallas_call(
        _post_kernel,
        grid=grid,
        in_specs=[
            pl.BlockSpec((1, ROWS_BLK, H), lambda i: (i // 5, i % 5, 0)),
            _rows((ROWS_BLK, H)),
            _full((H, H)),
            _full((H, H)), _full((1, H)),
            _full((H, 16)), _full((1, 16)),
        ],
        out_specs=_rows((ROWS_BLK, 16)),
        out_shape=jax.ShapeDtypeStruct((n, 16), jnp.float32),
    )(p1, s1, l1_wl, fc1_eff, fc1_b[None, :], fc2_w, fc2_b[None, :])

    return out


# speedup vs baseline: 6.2170x; 1.0131x over previous
import functools

import jax
import jax.numpy as jnp
from jax import lax
from jax.experimental import pallas as pl
from jax.experimental.pallas import tpu as pltpu
from jax.experimental.pallas import tpu_sc as plsc

N = 10000
E = 320000
D_IN = 128
H = 64

NC = 2
NS = 16
LANES = 16

ROWS_BLK = 1000


def _make_seg_sum(n_rows: int, d: int, rel: int, n_edges: int):
    nsh = n_rows // NC
    ept = n_edges // NS
    n_seg = 5
    seg = ept // n_seg
    assert seg * n_seg == ept and seg % LANES == 0
    cbuf_len = seg + 144
    rows_per_tile = 320
    nrp = rows_per_tile * NS
    assert nrp >= nsh
    zrows = 64
    n_zcopies = rows_per_tile // zrows
    garbage_row = nrp

    mesh = plsc.VectorSubcoreMesh(
        core_axis_name="c", subcore_axis_name="s",
        num_cores=NC, num_subcores=NS)

    @functools.partial(
        pl.kernel,
        out_type=jax.ShapeDtypeStruct((NC, nrp, d), jnp.float32),
        mesh=mesh,
        compiler_params=pltpu.CompilerParams(
            needs_layout_passes=False, use_tc_tiling_on_sc=False),
        scratch_types=dict(
            aggr=pltpu.VMEM_SHARED((nrp + 8, d), jnp.float32),
            src_v=pltpu.VMEM((seg,), jnp.int32),
            dst_v=pltpu.VMEM((seg,), jnp.int32),
            typ_v=pltpu.VMEM((seg,), jnp.int32),
            csrc=pltpu.VMEM((cbuf_len,), jnp.int32),
            cdst=pltpu.VMEM((cbuf_len,), jnp.int32),
            didx=pltpu.VMEM((2, 128), jnp.int32),
            sidx=pltpu.VMEM((2, 128), jnp.int32),
            gbuf=pltpu.VMEM((2, 128, d), jnp.float32),
            zbuf=pltpu.VMEM((zrows, d), jnp.float32),
            gsem=pltpu.SemaphoreType.DMA,
            ssem=pltpu.SemaphoreType.DMA,
        ),
    )
    def seg_sum(h_hbm, esrc_hbm, edst_hbm, et_hbm, out_hbm,
                aggr, src_v, dst_v, typ_v, csrc, cdst, didx, sidx, gbuf,
                zbuf, gsem, ssem):
        c = lax.axis_index("c")
        s = lax.axis_index("s")
        lanes_per_row = d // LANES
        lo = c * nsh
        lov = lax.broadcast_in_dim(lo, (LANES,), ())
        hiv = lov + nsh

        def zb(k, _):
            zbuf[k // lanes_per_row,
                 pl.ds((k % lanes_per_row) * LANES, LANES)] = (
                     jnp.zeros((LANES,), jnp.float32))
            return 0
        lax.fori_loop(0, zrows * lanes_per_row, zb, 0)
        row_lo = s * rows_per_tile
        for z in range(n_zcopies):
            pltpu.sync_copy(zbuf, aggr.at[pl.ds(row_lo + z * zrows, zrows)])

        plsc.subcore_barrier()

        for g in range(n_seg):
            ebase = s * ept + g * seg
            pltpu.sync_copy(esrc_hbm.at[pl.ds(ebase, seg)], src_v)
            pltpu.sync_copy(edst_hbm.at[pl.ds(ebase, seg)], dst_v)
            pltpu.sync_copy(et_hbm.at[pl.ds(ebase, seg)], typ_v)

            def cbody(i, off):
                b = i * LANES
                sv = src_v[pl.ds(b, LANES)]
                dv = dst_v[pl.ds(b, LANES)]
                tv = typ_v[pl.ds(b, LANES)]
                m = (tv == rel) & (sv >= lov) & (sv < hiv)
                mi = m.astype(jnp.int32)
                plsc.store_compressed(csrc.at[pl.ds(off, LANES)],
                                      sv - lov, mask=m)
                plsc.store_compressed(cdst.at[pl.ds(off, LANES)],
                                      dv, mask=m)
                return off + jnp.sum(mi)
            off = lax.fori_loop(0, seg // LANES, cbody, jnp.int32(0))

            for j in range(8):
                pidx = off + j * LANES + lax.iota(jnp.int32, 16)
                plsc.store_scatter(csrc, [pidx],
                                   jnp.full((LANES,), garbage_row, jnp.int32))
                plsc.store_scatter(cdst, [pidx],
                                   jnp.zeros((LANES,), jnp.int32))

            nch = (off + 127) // 128

            def pair_body(p, _):
                for b in range(2):
                    ch = p * 2 + b
                    @pl.when(ch < nch)
                    def _(b=b, ch=ch):
                        base = ch * 128
                        for j in range(8):
                            sidx[b, pl.ds(j * LANES, LANES)] = (
                                csrc[pl.ds(base + j * LANES, LANES)])
                            didx[b, pl.ds(j * LANES, LANES)] = (
                                cdst[pl.ds(base + j * LANES, LANES)])
                        pltpu.async_copy(h_hbm.at[didx.at[b]],
                                         gbuf.at[b], gsem)
                for b in range(2):
                    ch = p * 2 + b
                    @pl.when(ch < nch)
                    def _(b=b):
                        pltpu.make_async_copy(h_hbm.at[didx.at[b]],
                                              gbuf.at[b], gsem).wait()
                        pltpu.async_copy(gbuf.at[b], aggr.at[sidx.at[b]],
                                         ssem, add=True)
                for b in range(2):
                    ch = p * 2 + b
                    @pl.when(ch < nch)
                    def _(b=b):
                        pltpu.make_async_copy(gbuf.at[b],
                                              aggr.at[sidx.at[b]],
                                              ssem).wait()
                return 0
            lax.fori_loop(0, (nch + 1) // 2, pair_body, 0)

        plsc.subcore_barrier()

        pltpu.sync_copy(aggr.at[pl.ds(row_lo, rows_per_tile)],
                        out_hbm.at[c, pl.ds(row_lo, rows_per_tile)])

    return seg_sum


def _dot(a, b):
    return jnp.dot(a, b, preferred_element_type=jnp.float32)


def _full(shape):
    return pl.BlockSpec(shape, lambda i: (0,) * len(shape))


def _rows(shape):
    return pl.BlockSpec(shape, lambda i: (i,) + (0,) * (len(shape) - 1))


def _mlp_kernel(x_ref, w1_ref, b1_ref, w2_ref, b2_ref, w3_ref, b3_ref,
                w01_ref, b01_ref, h_ref, s0_ref):
    h1 = jnp.maximum(_dot(x_ref[...], w1_ref[...]) + b1_ref[...], 0.0)
    h2 = jnp.maximum(_dot(h1, w2_ref[...]) + b2_ref[...], 0.0)
    hh = _dot(h2, w3_ref[...]) + b3_ref[...]
    h_ref[...] = hh
    s0_ref[...] = _dot(hh, w01_ref[...]) + b01_ref[...]


def _mid_kernel(p0_ref, s0_ref, wl_ref, w11_ref, b11_ref, emb1_ref, s1_ref):
    aggr = p0_ref[0]
    emb1 = jnp.maximum(_dot(aggr, wl_ref[...]) + s0_ref[...], 0.0)
    emb1_ref[...] = emb1
    s1_ref[...] = _dot(emb1, w11_ref[...]) + b11_ref[...]


def _post_kernel(p1_ref, s1_ref, wl_ref, fc1w_ref, fc1b_ref, fc2w_ref,
                 fc2b_ref, out_ref):
    aggr = p1_ref[0]
    emb2 = jnp.maximum(_dot(aggr, wl_ref[...]) + s1_ref[...], 0.0)
    o1 = jnp.maximum(_dot(emb2, fc1w_ref[...]) + fc1b_ref[...], 0.0)
    o2 = _dot(o1, fc2w_ref[...]) + fc2b_ref[...]
    mx = jnp.max(o2, axis=1, keepdims=True)
    lse = jnp.log(jnp.sum(jnp.exp(o2 - mx), axis=1, keepdims=True)) + mx
    out_ref[...] = o2 - lse


def kernel(x, edge_index, edge_type,
           mlp_w1, mlp_b1, mlp_w2, mlp_b2, mlp_w3, mlp_b3,
           l0_w0, l0_b0, l0_wl, l0_bl, l0_w1, l0_b1,
           l1_w0, l1_b0, l1_wl, l1_bl, l1_w1, l1_b1,
           fc1_w, fc1_b, fc2_w, fc2_b):
    n = x.shape[0]
    grid = (n // ROWS_BLK,)

    w01 = l0_w0 + l0_w1
    b01 = (l0_b0 + l0_b1 + l0_bl)[None, :]
    w11 = l1_w0 + l1_w1
    b11 = (l1_b0 + l1_b1 + l1_bl)[None, :]
    fc1_eff = fc1_w[:H] + fc1_w[H:]

    h, s0 = pl.pallas_call(
        _mlp_kernel,
        grid=grid,
        in_specs=[
            _rows((ROWS_BLK, D_IN)),
            _full((D_IN, H)), _full((1, H)),
            _full((H, H)), _full((1, H)),
            _full((H, 2 * H)), _full((1, 2 * H)),
            _full((2 * H, H)), _full((1, H)),
        ],
        out_specs=[_rows((ROWS_BLK, 2 * H)), _rows((ROWS_BLK, H))],
        out_shape=[
            jax.ShapeDtypeStruct((n, 2 * H), jnp.float32),
            jax.ShapeDtypeStruct((n, H), jnp.float32),
        ],
    )(x, mlp_w1, mlp_b1[None, :], mlp_w2, mlp_b2[None, :],
      mlp_w3, mlp_b3[None, :], w01, b01)

    esrc = edge_index[0]
    edst = edge_index[1]
    p0 = _make_seg_sum(n, 2 * H, 2, edge_type.shape[0])(
        h, esrc, edst, edge_type)

    emb1, s1 = pl.pallas_call(
        _mid_kernel,
        grid=grid,
        in_specs=[
            pl.BlockSpec((1, ROWS_BLK, 2 * H), lambda i: (i // 5, i % 5, 0)),
            _rows((ROWS_BLK, H)),
            _full((2 * H, H)),
            _full((H, H)), _full((1, H)),
        ],
        out_specs=[_rows((ROWS_BLK, H)), _rows((ROWS_BLK, H))],
        out_shape=[
            jax.ShapeDtypeStruct((n, H), jnp.float32),
            jax.ShapeDtypeStruct((n, H), jnp.float32),
        ],
    )(p0, s0, l0_wl, w11, b11)

    p1 = _make_seg_sum(n, H, 3, edge_type.shape[0])(
        emb1, esrc, edst, edge_type)

    out = pl.pallas_call(
        _post_kernel,
        grid=grid,
        in_specs=[
            pl.BlockSpec((1, ROWS_BLK, H), lambda i: (i // 5, i % 5, 0)),
            _rows((ROWS_BLK, H)),
            _full((H, H)),
            _full((H, H)), _full((1, H)),
            _full((H, 16)), _full((1, 16)),
        ],
        out_specs=_rows((ROWS_BLK, 16)),
        out_shape=jax.ShapeDtypeStruct((n, 16), jnp.float32),
    )(p1, s1, l1_wl, fc1_eff, fc1_b[None, :], fc2_w, fc2_b[None, :])

    return out


# speedup vs baseline: 8.4436x; 1.3582x over previous
import functools

import jax
import jax.numpy as jnp
from jax import lax
from jax.experimental import pallas as pl
from jax.experimental.pallas import tpu as pltpu
from jax.experimental.pallas import tpu_sc as plsc

N = 10000
E = 320000
D_IN = 128
H = 64

NC = 2
NS = 16
LANES = 16

ROWS_BLK = 1000


def _make_seg_sum(n_rows: int, d: int, rel: int, n_edges: int):
    nsh = n_rows // NC
    ept = n_edges // NS
    n_seg = 5
    seg = ept // n_seg
    assert seg * n_seg == ept and seg % LANES == 0
    cbuf_len = seg + 144
    rows_per_tile = 320
    nrp = rows_per_tile * NS
    assert nrp >= nsh
    zrows = 64
    n_zcopies = rows_per_tile // zrows
    garbage_row = nrp

    mesh = plsc.VectorSubcoreMesh(
        core_axis_name="c", subcore_axis_name="s",
        num_cores=NC, num_subcores=NS)

    @functools.partial(
        pl.kernel,
        out_type=jax.ShapeDtypeStruct((NC, nrp, d), jnp.float32),
        mesh=mesh,
        compiler_params=pltpu.CompilerParams(
            needs_layout_passes=False, use_tc_tiling_on_sc=False),
        scratch_types=dict(
            aggr=pltpu.VMEM_SHARED((nrp + 8, d), jnp.float32),
            src_v=pltpu.VMEM((seg,), jnp.int32),
            dst_v=pltpu.VMEM((seg,), jnp.int32),
            typ_v=pltpu.VMEM((seg,), jnp.int32),
            csrc=pltpu.VMEM((cbuf_len,), jnp.int32),
            cdst=pltpu.VMEM((cbuf_len,), jnp.int32),
            didx=pltpu.VMEM((2, 128), jnp.int32),
            sidx=pltpu.VMEM((2, 128), jnp.int32),
            gbuf=pltpu.VMEM((2, 128, d), jnp.float32),
            zbuf=pltpu.VMEM((zrows, d), jnp.float32),
            gsem=pltpu.SemaphoreType.DMA,
            ssem=pltpu.SemaphoreType.DMA,
        ),
    )
    def seg_sum(h_hbm, esrc_hbm, edst_hbm, et_hbm, out_hbm,
                aggr, src_v, dst_v, typ_v, csrc, cdst, didx, sidx, gbuf,
                zbuf, gsem, ssem):
        c = lax.axis_index("c")
        s = lax.axis_index("s")
        lanes_per_row = d // LANES
        lo = c * nsh
        lov = lax.broadcast_in_dim(lo, (LANES,), ())
        hiv = lov + nsh

        def zb(k, _):
            zbuf[k // lanes_per_row,
                 pl.ds((k % lanes_per_row) * LANES, LANES)] = (
                     jnp.zeros((LANES,), jnp.float32))
            return 0
        lax.fori_loop(0, zrows * lanes_per_row, zb, 0)
        row_lo = s * rows_per_tile
        for z in range(n_zcopies):
            pltpu.sync_copy(zbuf, aggr.at[pl.ds(row_lo + z * zrows, zrows)])

        plsc.subcore_barrier()

        for g in range(n_seg):
            ebase = s * ept + g * seg
            pltpu.sync_copy(esrc_hbm.at[pl.ds(ebase, seg)], src_v)
            pltpu.sync_copy(edst_hbm.at[pl.ds(ebase, seg)], dst_v)
            pltpu.sync_copy(et_hbm.at[pl.ds(ebase, seg)], typ_v)

            def cbody(i, off):
                b = i * LANES
                sv = src_v[pl.ds(b, LANES)]
                dv = dst_v[pl.ds(b, LANES)]
                tv = typ_v[pl.ds(b, LANES)]
                m = (tv == rel) & (sv >= lov) & (sv < hiv)
                mi = m.astype(jnp.int32)
                plsc.store_compressed(csrc.at[pl.ds(off, LANES)],
                                      sv - lov, mask=m)
                plsc.store_compressed(cdst.at[pl.ds(off, LANES)],
                                      dv, mask=m)
                return off + jnp.sum(mi)
            off = lax.fori_loop(0, seg // LANES, cbody, jnp.int32(0))

            for j in range(8):
                pidx = off + j * LANES + lax.iota(jnp.int32, 16)
                plsc.store_scatter(csrc, [pidx],
                                   jnp.full((LANES,), garbage_row, jnp.int32))
                plsc.store_scatter(cdst, [pidx],
                                   jnp.zeros((LANES,), jnp.int32))

            nch = (off + 127) // 128

            def pair_body(p, _):
                for b in range(2):
                    ch = p * 2 + b
                    @pl.when(ch < nch)
                    def _(b=b, ch=ch):
                        base = ch * 128
                        for j in range(8):
                            sidx[b, pl.ds(j * LANES, LANES)] = (
                                csrc[pl.ds(base + j * LANES, LANES)])
                            didx[b, pl.ds(j * LANES, LANES)] = (
                                cdst[pl.ds(base + j * LANES, LANES)])
                        pltpu.async_copy(h_hbm.at[didx.at[b]],
                                         gbuf.at[b], gsem)
                for b in range(2):
                    ch = p * 2 + b
                    @pl.when(ch < nch)
                    def _(b=b):
                        pltpu.make_async_copy(h_hbm.at[didx.at[b]],
                                              gbuf.at[b], gsem).wait()
                        pltpu.async_copy(gbuf.at[b], aggr.at[sidx.at[b]],
                                         ssem, add=True)
                for b in range(2):
                    ch = p * 2 + b
                    @pl.when(ch < nch)
                    def _(b=b):
                        pltpu.make_async_copy(gbuf.at[b],
                                              aggr.at[sidx.at[b]],
                                              ssem).wait()
                return 0
            lax.fori_loop(0, (nch + 1) // 2, pair_body, 0)

        plsc.subcore_barrier()

        pltpu.sync_copy(aggr.at[pl.ds(row_lo, rows_per_tile)],
                        out_hbm.at[c, pl.ds(row_lo, rows_per_tile)])

    return seg_sum


def _dot(a, b):
    return jnp.dot(a, b, preferred_element_type=jnp.float32)


def _full(shape):
    return pl.BlockSpec(shape, lambda i: (0,) * len(shape))


def _rows(shape):
    return pl.BlockSpec(shape, lambda i: (i,) + (0,) * (len(shape) - 1))


def _mlp_kernel(x_ref, w1_ref, b1_ref, w2_ref, b2_ref, w3_ref, b3_ref,
                w01_ref, b01_ref, wl0_ref, hw0_ref, s0_ref):
    h1 = jnp.maximum(_dot(x_ref[...], w1_ref[...]) + b1_ref[...], 0.0)
    h2 = jnp.maximum(_dot(h1, w2_ref[...]) + b2_ref[...], 0.0)
    hh = _dot(h2, w3_ref[...]) + b3_ref[...]
    hw0_ref[...] = _dot(hh, wl0_ref[...])
    s0_ref[...] = _dot(hh, w01_ref[...]) + b01_ref[...]


def _mid_kernel(p0_ref, s0_ref, w11_ref, b11_ref, wl1_ref, hw1_ref, s1_ref):
    emb1 = jnp.maximum(p0_ref[0] + s0_ref[...], 0.0)
    hw1_ref[...] = _dot(emb1, wl1_ref[...])
    s1_ref[...] = _dot(emb1, w11_ref[...]) + b11_ref[...]


def _post_kernel(p1_ref, s1_ref, fc1w_ref, fc1b_ref, fc2w_ref,
                 fc2b_ref, out_ref):
    emb2 = jnp.maximum(p1_ref[0] + s1_ref[...], 0.0)
    o1 = jnp.maximum(_dot(emb2, fc1w_ref[...]) + fc1b_ref[...], 0.0)
    o2 = _dot(o1, fc2w_ref[...]) + fc2b_ref[...]
    mx = jnp.max(o2, axis=1, keepdims=True)
    lse = jnp.log(jnp.sum(jnp.exp(o2 - mx), axis=1, keepdims=True)) + mx
    out_ref[...] = o2 - lse


def kernel(x, edge_index, edge_type,
           mlp_w1, mlp_b1, mlp_w2, mlp_b2, mlp_w3, mlp_b3,
           l0_w0, l0_b0, l0_wl, l0_bl, l0_w1, l0_b1,
           l1_w0, l1_b0, l1_wl, l1_bl, l1_w1, l1_b1,
           fc1_w, fc1_b, fc2_w, fc2_b):
    n = x.shape[0]
    grid = (n // ROWS_BLK,)

    w01 = l0_w0 + l0_w1
    b01 = (l0_b0 + l0_b1 + l0_bl)[None, :]
    w11 = l1_w0 + l1_w1
    b11 = (l1_b0 + l1_b1 + l1_bl)[None, :]
    fc1_eff = fc1_w[:H] + fc1_w[H:]

    hw0, s0 = pl.pallas_call(
        _mlp_kernel,
        grid=grid,
        in_specs=[
            _rows((ROWS_BLK, D_IN)),
            _full((D_IN, H)), _full((1, H)),
            _full((H, H)), _full((1, H)),
            _full((H, 2 * H)), _full((1, 2 * H)),
            _full((2 * H, H)), _full((1, H)),
            _full((2 * H, H)),
        ],
        out_specs=[_rows((ROWS_BLK, H)), _rows((ROWS_BLK, H))],
        out_shape=[
            jax.ShapeDtypeStruct((n, H), jnp.float32),
            jax.ShapeDtypeStruct((n, H), jnp.float32),
        ],
    )(x, mlp_w1, mlp_b1[None, :], mlp_w2, mlp_b2[None, :],
      mlp_w3, mlp_b3[None, :], w01, b01, l0_wl)

    esrc = edge_index[0]
    edst = edge_index[1]
    p0 = _make_seg_sum(n, H, 2, edge_type.shape[0])(
        hw0, esrc, edst, edge_type)

    hw1, s1 = pl.pallas_call(
        _mid_kernel,
        grid=grid,
        in_specs=[
            pl.BlockSpec((1, ROWS_BLK, H), lambda i: (i // 5, i % 5, 0)),
            _rows((ROWS_BLK, H)),
            _full((H, H)), _full((1, H)),
            _full((H, H)),
        ],
        out_specs=[_rows((ROWS_BLK, H)), _rows((ROWS_BLK, H))],
        out_shape=[
            jax.ShapeDtypeStruct((n, H), jnp.float32),
            jax.ShapeDtypeStruct((n, H), jnp.float32),
        ],
    )(p0, s0, w11, b11, l1_wl)

    p1 = _make_seg_sum(n, H, 3, edge_type.shape[0])(
        hw1, esrc, edst, edge_type)

    out = pl.pallas_call(
        _post_kernel,
        grid=grid,
        in_specs=[
            pl.BlockSpec((1, ROWS_BLK, H), lambda i: (i // 5, i % 5, 0)),
            _rows((ROWS_BLK, H)),
            _full((H, H)), _full((1, H)),
            _full((H, 16)), _full((1, 16)),
        ],
        out_specs=_rows((ROWS_BLK, 16)),
        out_shape=jax.ShapeDtypeStruct((n, 16), jnp.float32),
    )(p1, s1, fc1_eff, fc1_b[None, :], fc2_w, fc2_b[None, :])

    return out


# speedup vs baseline: 23.3253x; 2.7625x over previous
import functools

import jax
import jax.numpy as jnp
from jax import lax
from jax.experimental import pallas as pl
from jax.experimental.pallas import tpu as pltpu
from jax.experimental.pallas import tpu_sc as plsc

N = 10000
E = 320000
D_IN = 128
H = 64

NC = 2
NS = 16
LANES = 16

ROWS_BLK = 5000
BPH = 1


NSH = N // NC
ROWS_PER_TILE = 320
NRP = ROWS_PER_TILE * NS
GARBAGE_ROW = NRP
CAP2 = 20480
WBLK = 2048


def _make_precompact(n_edges: int):
    ept = n_edges // NS
    n_seg = 5
    seg = ept // n_seg
    assert seg * n_seg == ept and seg % (2 * LANES) == 0

    mesh = plsc.VectorSubcoreMesh(
        core_axis_name="c", subcore_axis_name="s",
        num_cores=NC, num_subcores=NS)

    @functools.partial(
        pl.kernel,
        out_type=[
            jax.ShapeDtypeStruct((2, NC, NS, CAP2), jnp.int32),
            jax.ShapeDtypeStruct((2, NC, NS, CAP2), jnp.int32),
            jax.ShapeDtypeStruct((2, NC, NS, 16), jnp.int32),
        ],
        mesh=mesh,
        compiler_params=pltpu.CompilerParams(
            needs_layout_passes=False, use_tc_tiling_on_sc=False),
        scratch_types=dict(
            src_v=pltpu.VMEM((seg,), jnp.int32),
            dst_v=pltpu.VMEM((seg,), jnp.int32),
            typ_v=pltpu.VMEM((seg,), jnp.int32),
            cs2=pltpu.VMEM((CAP2,), jnp.int32),
            cd2=pltpu.VMEM((CAP2,), jnp.int32),
            cs3=pltpu.VMEM((CAP2,), jnp.int32),
            cd3=pltpu.VMEM((CAP2,), jnp.int32),
            cntb=pltpu.VMEM((16,), jnp.int32),
        ),
    )
    def precompact(esrc_hbm, edst_hbm, et_hbm, lsrc_hbm, ldst_hbm, ncnt_hbm,
                   src_v, dst_v, typ_v, cs2, cd2, cs3, cd3, cntb):
        c = lax.axis_index("c")
        s = lax.axis_index("s")
        lov = lax.broadcast_in_dim(c * NSH, (LANES,), ())
        hiv = lov + NSH
        cs = (cs2, cs3)
        cd = (cd2, cd3)

        offs = (jnp.int32(0), jnp.int32(0))
        for g in range(n_seg):
            ebase = s * ept + g * seg
            pltpu.sync_copy(esrc_hbm.at[pl.ds(ebase, seg)], src_v)
            pltpu.sync_copy(edst_hbm.at[pl.ds(ebase, seg)], dst_v)
            pltpu.sync_copy(et_hbm.at[pl.ds(ebase, seg)], typ_v)

            def cbody(i, offs):
                off2, off3 = offs
                b = i * LANES
                sv = src_v[pl.ds(b, LANES)]
                dv = dst_v[pl.ds(b, LANES)]
                tv = typ_v[pl.ds(b, LANES)]
                inh = (sv >= lov) & (sv < hiv)
                slv = sv - lov
                m2 = (tv == 2) & inh
                m3 = (tv == 3) & inh
                plsc.store_compressed(cs2.at[pl.ds(off2, LANES)],
                                      slv, mask=m2)
                plsc.store_compressed(cd2.at[pl.ds(off2, LANES)],
                                      dv, mask=m2)
                plsc.store_compressed(cs3.at[pl.ds(off3, LANES)],
                                      slv, mask=m3)
                plsc.store_compressed(cd3.at[pl.ds(off3, LANES)],
                                      dv, mask=m3)
                return (off2 + jnp.sum(m2.astype(jnp.int32)),
                        off3 + jnp.sum(m3.astype(jnp.int32)))
            offs = lax.fori_loop(0, seg // LANES, cbody, offs)

        for r in range(2):
            off = offs[r]
            for j in range(8):
                pidx = off + j * LANES + lax.iota(jnp.int32, 16)
                plsc.store_scatter(cs[r], [pidx],
                                   jnp.full((LANES,), GARBAGE_ROW, jnp.int32))
                plsc.store_scatter(cd[r], [pidx],
                                   jnp.zeros((LANES,), jnp.int32))
            nch = (off + 127) // 128
            cntb[pl.ds(0, LANES)] = lax.broadcast_in_dim(nch, (LANES,), ())
            pltpu.sync_copy(cntb, ncnt_hbm.at[r, c, s])
            nblk = (nch * 128 + WBLK - 1) // WBLK

            def wbody(b, _, r=r):
                pltpu.sync_copy(cs[r].at[pl.ds(b * WBLK, WBLK)],
                                lsrc_hbm.at[r, c, s, pl.ds(b * WBLK, WBLK)])
                pltpu.sync_copy(cd[r].at[pl.ds(b * WBLK, WBLK)],
                                ldst_hbm.at[r, c, s, pl.ds(b * WBLK, WBLK)])
                return 0
            lax.fori_loop(0, nblk, wbody, 0)

    return precompact


def _make_agg(d: int, relidx: int):
    zrows = 64
    n_zcopies = ROWS_PER_TILE // zrows

    mesh = plsc.VectorSubcoreMesh(
        core_axis_name="c", subcore_axis_name="s",
        num_cores=NC, num_subcores=NS)

    @functools.partial(
        pl.kernel,
        out_type=jax.ShapeDtypeStruct((NC, NRP, d), jnp.float32),
        mesh=mesh,
        compiler_params=pltpu.CompilerParams(
            needs_layout_passes=False, use_tc_tiling_on_sc=False),
        scratch_types=dict(
            aggr=pltpu.VMEM_SHARED((NRP + 8, d), jnp.float32),
            hsh=pltpu.VMEM_SHARED((N, d), jnp.float32),
            didx=pltpu.VMEM((4, 128), jnp.int32),
            sidx=pltpu.VMEM((4, 128), jnp.int32),
            gbuf=pltpu.VMEM((4, 128, d), jnp.float32),
            zbuf=pltpu.VMEM((64, d), jnp.float32),
            cnt_v=pltpu.VMEM((16,), jnp.int32),
            isem=pltpu.SemaphoreType.DMA,
            gsem=pltpu.SemaphoreType.DMA,
            ssem=pltpu.SemaphoreType.DMA,
        ),
    )
    def agg(h_hbm, lsrc_hbm, ldst_hbm, ncnt_hbm, out_hbm,
            aggr, hsh, didx, sidx, gbuf, zbuf, cnt_v, isem, gsem, ssem):
        c = lax.axis_index("c")
        s = lax.axis_index("s")
        lanes_per_row = d // LANES

        def zb(k, _):
            zbuf[k // lanes_per_row,
                 pl.ds((k % lanes_per_row) * LANES, LANES)] = (
                     jnp.zeros((LANES,), jnp.float32))
            return 0
        lax.fori_loop(0, zrows * lanes_per_row, zb, 0)
        row_lo = s * ROWS_PER_TILE
        for z in range(n_zcopies):
            pltpu.sync_copy(zbuf, aggr.at[pl.ds(row_lo + z * zrows, zrows)])

        hrows = N // NS
        hlo = s * hrows
        pltpu.sync_copy(h_hbm.at[pl.ds(hlo, hrows)], hsh.at[pl.ds(hlo, hrows)])

        pltpu.sync_copy(ncnt_hbm.at[relidx, c, s], cnt_v)
        nch = jnp.max(cnt_v[pl.ds(0, LANES)])

        plsc.subcore_barrier()

        def ring_body(p, _):
            for b in range(4):
                ch = p * 4 + b
                @pl.when(ch < nch)
                def _(b=b, ch=ch):
                    base = ch * 128
                    pltpu.async_copy(
                        lsrc_hbm.at[relidx, c, s, pl.ds(base, 128)],
                        sidx.at[b], isem)
                    pltpu.async_copy(
                        ldst_hbm.at[relidx, c, s, pl.ds(base, 128)],
                        didx.at[b], isem)
            for b in range(4):
                ch = p * 4 + b
                @pl.when(ch < nch)
                def _(b=b, ch=ch):
                    base = ch * 128
                    pltpu.make_async_copy(
                        lsrc_hbm.at[relidx, c, s, pl.ds(base, 128)],
                        sidx.at[b], isem).wait()
                    pltpu.make_async_copy(
                        ldst_hbm.at[relidx, c, s, pl.ds(base, 128)],
                        didx.at[b], isem).wait()
                    pltpu.async_copy(hsh.at[didx.at[b]], gbuf.at[b], gsem)
            for b in range(4):
                ch = p * 4 + b
                @pl.when(ch < nch)
                def _(b=b):
                    pltpu.make_async_copy(hsh.at[didx.at[b]],
                                          gbuf.at[b], gsem).wait()
                    pltpu.async_copy(gbuf.at[b], aggr.at[sidx.at[b]],
                                     ssem, add=True)
            for b in range(4):
                ch = p * 4 + b
                @pl.when(ch < nch)
                def _(b=b):
                    pltpu.make_async_copy(gbuf.at[b],
                                          aggr.at[sidx.at[b]],
                                          ssem).wait()
            return 0
        lax.fori_loop(0, (nch + 3) // 4, ring_body, 0)

        plsc.subcore_barrier()

        pltpu.sync_copy(aggr.at[pl.ds(row_lo, ROWS_PER_TILE)],
                        out_hbm.at[c, pl.ds(row_lo, ROWS_PER_TILE)])

    return agg


def _dot(a, b):
    return jnp.dot(a, b, preferred_element_type=jnp.float32)


def _full(shape):
    return pl.BlockSpec(shape, lambda i: (0,) * len(shape))


def _rows(shape):
    return pl.BlockSpec(shape, lambda i: (i,) + (0,) * (len(shape) - 1))


def _mlp_kernel(x_ref, w1_ref, b1_ref, w2_ref, b2_ref, w3_ref, b3_ref,
                w01_ref, b01_ref, wl0_ref, hw0_ref, s0_ref):
    h1 = jnp.maximum(_dot(x_ref[...], w1_ref[...]) + b1_ref[...], 0.0)
    h2 = jnp.maximum(_dot(h1, w2_ref[...]) + b2_ref[...], 0.0)
    hh = _dot(h2, w3_ref[...]) + b3_ref[...]
    hw0_ref[...] = _dot(hh, wl0_ref[...])
    s0_ref[...] = _dot(hh, w01_ref[...]) + b01_ref[...]


def _mid_kernel(p0_ref, s0_ref, w11_ref, b11_ref, wl1_ref, hw1_ref, s1_ref):
    emb1 = jnp.maximum(p0_ref[0] + s0_ref[...], 0.0)
    hw1_ref[...] = _dot(emb1, wl1_ref[...])
    s1_ref[...] = _dot(emb1, w11_ref[...]) + b11_ref[...]


def _post_kernel(p1_ref, s1_ref, fc1w_ref, fc1b_ref, fc2w_ref,
                 fc2b_ref, out_ref):
    emb2 = jnp.maximum(p1_ref[0] + s1_ref[...], 0.0)
    o1 = jnp.maximum(_dot(emb2, fc1w_ref[...]) + fc1b_ref[...], 0.0)
    o2 = _dot(o1, fc2w_ref[...]) + fc2b_ref[...]
    mx = jnp.max(o2, axis=1, keepdims=True)
    lse = jnp.log(jnp.sum(jnp.exp(o2 - mx), axis=1, keepdims=True)) + mx
    out_ref[...] = o2 - lse


def kernel(x, edge_index, edge_type,
           mlp_w1, mlp_b1, mlp_w2, mlp_b2, mlp_w3, mlp_b3,
           l0_w0, l0_b0, l0_wl, l0_bl, l0_w1, l0_b1,
           l1_w0, l1_b0, l1_wl, l1_bl, l1_w1, l1_b1,
           fc1_w, fc1_b, fc2_w, fc2_b):
    n = x.shape[0]
    grid = (n // ROWS_BLK,)

    w01 = l0_w0 + l0_w1
    b01 = (l0_b0 + l0_b1 + l0_bl)[None, :]
    w11 = l1_w0 + l1_w1
    b11 = (l1_b0 + l1_b1 + l1_bl)[None, :]
    fc1_eff = fc1_w[:H] + fc1_w[H:]

    hw0, s0 = pl.pallas_call(
        _mlp_kernel,
        grid=grid,
        in_specs=[
            _rows((ROWS_BLK, D_IN)),
            _full((D_IN, H)), _full((1, H)),
            _full((H, H)), _full((1, H)),
            _full((H, 2 * H)), _full((1, 2 * H)),
            _full((2 * H, H)), _full((1, H)),
            _full((2 * H, H)),
        ],
        out_specs=[_rows((ROWS_BLK, H)), _rows((ROWS_BLK, H))],
        out_shape=[
            jax.ShapeDtypeStruct((n, H), jnp.float32),
            jax.ShapeDtypeStruct((n, H), jnp.float32),
        ],
    )(x, mlp_w1, mlp_b1[None, :], mlp_w2, mlp_b2[None, :],
      mlp_w3, mlp_b3[None, :], w01, b01, l0_wl)

    esrc = edge_index[0]
    edst = edge_index[1]
    lsrc, ldst, ncnt = _make_precompact(edge_type.shape[0])(
        esrc, edst, edge_type)
    p0 = _make_agg(H, 0)(hw0, lsrc, ldst, ncnt)

    hw1, s1 = pl.pallas_call(
        _mid_kernel,
        grid=grid,
        in_specs=[
            pl.BlockSpec((1, ROWS_BLK, H), lambda i: (i // BPH, i % BPH, 0)),
            _rows((ROWS_BLK, H)),
            _full((H, H)), _full((1, H)),
            _full((H, H)),
        ],
        out_specs=[_rows((ROWS_BLK, H)), _rows((ROWS_BLK, H))],
        out_shape=[
            jax.ShapeDtypeStruct((n, H), jnp.float32),
            jax.ShapeDtypeStruct((n, H), jnp.float32),
        ],
    )(p0, s0, w11, b11, l1_wl)

    p1 = _make_agg(H, 1)(hw1, lsrc, ldst, ncnt)

    out = pl.pallas_call(
        _post_kernel,
        grid=grid,
        in_specs=[
            pl.BlockSpec((1, ROWS_BLK, H), lambda i: (i // BPH, i % BPH, 0)),
            _rows((ROWS_BLK, H)),
            _full((H, H)), _full((1, H)),
            _full((H, 16)), _full((1, 16)),
        ],
        out_specs=_rows((ROWS_BLK, 16)),
        out_shape=jax.ShapeDtypeStruct((n, 16), jnp.float32),
    )(p1, s1, fc1_eff, fc1_b[None, :], fc2_w, fc2_b[None, :])

    return out
